# baseline (speedup 1.0000x reference)
import functools

import jax
import jax.numpy as jnp
from jax import lax
from jax.experimental import pallas as pl
from jax.experimental.pallas import tpu as pltpu
from jax.experimental.pallas import tpu_sc as plsc

B, T, D = 1, 2048, 1024
E, K = 8, 2
INTER = 512
LIMIT = 10.0

NT = 8
TBLK = T // NT
AK = T * K
BLK = 128
CAP = AK + E * BLK
NBLK = CAP // BLK
CAPB = 48
NC, NS, L = 2, 16, 16
NW = NC * NS
RPW = CAP // NW
RCH = RPW // 2
TRASH = AK


def _route_body(flat_ref, gate_ref, s1_ref, s2_ref, s3_ref,
                y_ref, eid_ref, w_ref, hist_ref):
    t = pl.program_id(0)
    xb = flat_ref[...]
    s = jnp.dot(xb, gate_ref[...], preferred_element_type=jnp.float32)
    scores = jnp.sqrt(jax.nn.softplus(s))
    iota = lax.broadcasted_iota(jnp.int32, (TBLK, E), 1)
    m1 = jnp.max(scores, axis=1, keepdims=True)
    idx1 = jnp.min(jnp.where(scores == m1, iota, E), axis=1, keepdims=True)
    mask1 = iota == idx1
    scores2 = jnp.where(mask1, -jnp.inf, scores)
    m2 = jnp.max(scores2, axis=1, keepdims=True)
    idx2 = jnp.min(jnp.where(scores2 == m2, iota, E), axis=1, keepdims=True)
    mask2 = iota == idx2
    denom = jnp.maximum(m1 + m2, 1e-6)
    eid_ref[...] = jnp.concatenate([idx1, idx2], axis=1)
    w_ref[...] = jnp.concatenate([m1 / denom, m2 / denom], axis=1)

    cnt = (jnp.sum(mask1.astype(jnp.int32), axis=0, keepdims=True)
           + jnp.sum(mask2.astype(jnp.int32), axis=0, keepdims=True))
    cnt16 = jnp.concatenate([cnt, jnp.zeros((1, E), jnp.int32)], axis=1)

    @pl.when(t == 0)
    def _():
        hist_ref[...] = cnt16

    @pl.when(t != 0)
    def _():
        hist_ref[...] += cnt16

    g = jnp.dot(xb, s1_ref[...], preferred_element_type=jnp.float32)
    u = jnp.dot(xb, s3_ref[...], preferred_element_type=jnp.float32)
    g = jnp.minimum(g, LIMIT)
    u = jnp.clip(u, -LIMIT, LIMIT)
    h = (g * jax.nn.sigmoid(g)) * u
    y_ref[...] = jnp.dot(h, s2_ref[...], preferred_element_type=jnp.float32)


def _route(flat, gate_w, sw1, sw2, sw3):
    return pl.pallas_call(
        _route_body,
        grid=(NT,),
        in_specs=[
            pl.BlockSpec((TBLK, D), lambda t: (t, 0)),
            pl.BlockSpec((D, E), lambda t: (0, 0)),
            pl.BlockSpec((D, INTER), lambda t: (0, 0)),
            pl.BlockSpec((INTER, D), lambda t: (0, 0)),
            pl.BlockSpec((D, INTER), lambda t: (0, 0)),
        ],
        out_specs=[
            pl.BlockSpec((TBLK, D), lambda t: (t, 0)),
            pl.BlockSpec((TBLK, K), lambda t: (t, 0)),
            pl.BlockSpec((TBLK, K), lambda t: (t, 0)),
            pl.BlockSpec((1, 2 * E), lambda t: (0, 0)),
        ],
        out_shape=[
            jax.ShapeDtypeStruct((T, D), jnp.float32),
            jax.ShapeDtypeStruct((T, K), jnp.int32),
            jax.ShapeDtypeStruct((T, K), jnp.float32),
            jax.ShapeDtypeStruct((1, 2 * E), jnp.int32),
        ],
    )(flat, gate_w, sw1, sw2, sw3)


def _sort_body(eid_hbm, w_hbm, hist_hbm, perm_hbm, dst_hbm, wsort_hbm, bexp_hbm,
               eid_v, w_v, hist_v, base_v, perm_v, dst_v, wsort_v, bexp_v):
    cid = lax.axis_index("c")
    sid = lax.axis_index("s")

    @pl.when((cid == 0) & (sid == 0))
    def _():
        pltpu.sync_copy(eid_hbm, eid_v)
        pltpu.sync_copy(w_hbm, w_v)
        pltpu.sync_copy(hist_hbm, hist_v)

        lane = lax.iota(jnp.int32, L)
        cnt = hist_v[...]
        nb = (cnt + (BLK - 1)) >> 7
        csum = plsc.cumsum(nb)
        pstart_blk = csum - nb
        base0 = pstart_blk * BLK
        base_v[...] = base0

        for i in range(CAPB // L):
            b_ids = i * L + lane
            acc = jnp.zeros((L,), jnp.int32)
            for e in range(E):
                pe = jnp.max(jnp.where(lane == e, csum, -1))
                acc = acc + jnp.where(b_ids >= pe, 1, 0)
            bexp_v[pl.ds(i * L, L)] = jnp.minimum(acc, E - 1)

        def init_body(i, c):
            perm_v[pl.ds(i * L, L)] = jnp.zeros((L,), jnp.int32)
            dst_v[pl.ds(i * L, L)] = jnp.full((L,), TRASH, jnp.int32)
            wsort_v[pl.ds(i * L, L)] = jnp.zeros((L,), jnp.float32)
            return c

        lax.fori_loop(0, CAP // L, init_body, 0)

        def s_body(i, c):
            a0 = i * L
            ev = eid_v[pl.ds(a0, L)]
            wv = w_v[pl.ds(a0, L)]
            ev_s, lane_s = plsc.sort_key_val(ev, lane)
            prev = ev_s.at[jnp.maximum(lane - 1, 0)].get(mode="promise_in_bounds")
            st = jnp.where((lane == 0) | (ev_s != prev), lane, 0)
            rank = lane - plsc.cummax(st)
            pos = plsc.load_gather(base_v, [ev_s]) + rank
            aid_s = a0 + lane_s
            tok_s = lax.shift_right_logical(aid_s, 1)
            w_s = wv.at[lane_s].get(mode="promise_in_bounds")
            plsc.store_scatter(perm_v, [pos], tok_s)
            plsc.store_scatter(dst_v, [pos], (aid_s & 1) * T + tok_s)
            plsc.store_scatter(wsort_v, [pos], w_s)
            nxt = ev_s.at[jnp.minimum(lane + 1, L - 1)].get(mode="promise_in_bounds")
            en = (lane == L - 1) | (ev_s != nxt)
            plsc.store_scatter(base_v, [ev_s], pos + 1, mask=en)
            return c

        lax.fori_loop(0, AK // L, s_body, 0)

        pltpu.sync_copy(perm_v, perm_hbm)
        pltpu.sync_copy(dst_v, dst_hbm)
        pltpu.sync_copy(wsort_v, wsort_hbm)
        pltpu.sync_copy(bexp_v, bexp_hbm)


_sort = functools.partial(
    pl.kernel,
    _sort_body,
    out_type=(
        jax.ShapeDtypeStruct((CAP,), jnp.int32),
        jax.ShapeDtypeStruct((CAP,), jnp.int32),
        jax.ShapeDtypeStruct((CAP,), jnp.float32),
        jax.ShapeDtypeStruct((CAPB,), jnp.int32),
    ),
    mesh=plsc.VectorSubcoreMesh(core_axis_name="c", subcore_axis_name="s", num_cores=NC, num_subcores=NS),
    compiler_params=pltpu.CompilerParams(needs_layout_passes=False),
    scratch_types=[
        pltpu.VMEM((AK,), jnp.int32),
        pltpu.VMEM((AK,), jnp.float32),
        pltpu.VMEM((L,), jnp.int32),
        pltpu.VMEM((L,), jnp.int32),
        pltpu.VMEM((CAP,), jnp.int32),
        pltpu.VMEM((CAP,), jnp.int32),
        pltpu.VMEM((CAP,), jnp.float32),
        pltpu.VMEM((CAPB,), jnp.int32),
    ],
)()


def _gather_body(flat_hbm, perm_hbm, out_hbm, idx_v, rows_v, sem):
    wid = lax.axis_index("s") * NC + lax.axis_index("c")
    base = wid * RPW
    for c in range(RPW // RCH):
        off = base + c * RCH
        pltpu.sync_copy(perm_hbm.at[pl.ds(off, RCH)], idx_v)
        pltpu.async_copy(flat_hbm.at[idx_v], rows_v, sem).wait()
        pltpu.sync_copy(rows_v, out_hbm.at[pl.ds(off, RCH)])


_gather = functools.partial(
    pl.kernel,
    _gather_body,
    out_type=jax.ShapeDtypeStruct((CAP, D), jnp.float32),
    mesh=plsc.VectorSubcoreMesh(core_axis_name="c", subcore_axis_name="s", num_cores=NC, num_subcores=NS),
    compiler_params=pltpu.CompilerParams(needs_layout_passes=False),
    scratch_types=[
        pltpu.VMEM((RCH,), jnp.int32),
        pltpu.VMEM((RCH, D), jnp.float32),
        pltpu.SemaphoreType.DMA,
    ],
)()


def _ffn_body(bexp_ref, x_ref, w1_ref, w3_ref, w2_ref, ws_ref, out_ref):
    xb = x_ref[...]
    g = jnp.dot(xb, w1_ref[0], preferred_element_type=jnp.float32)
    u = jnp.dot(xb, w3_ref[0], preferred_element_type=jnp.float32)
    g = jnp.minimum(g, LIMIT)
    u = jnp.clip(u, -LIMIT, LIMIT)
    h = (g * jax.nn.sigmoid(g)) * u
    out_ref[...] = ws_ref[...] * jnp.dot(h, w2_ref[0], preferred_element_type=jnp.float32)


def _ffn(bexp, gathered, W1, W3, W2, wsort2):
    grid_spec = pltpu.PrefetchScalarGridSpec(
        num_scalar_prefetch=1,
        grid=(NBLK,),
        in_specs=[
            pl.BlockSpec((BLK, D), lambda b, be: (b, 0)),
            pl.BlockSpec((1, D, INTER), lambda b, be: (be[b], 0, 0)),
            pl.BlockSpec((1, D, INTER), lambda b, be: (be[b], 0, 0)),
            pl.BlockSpec((1, INTER, D), lambda b, be: (be[b], 0, 0)),
            pl.BlockSpec((BLK, 1), lambda b, be: (b, 0)),
        ],
        out_specs=pl.BlockSpec((BLK, D), lambda b, be: (b, 0)),
    )
    return pl.pallas_call(
        _ffn_body,
        grid_spec=grid_spec,
        out_shape=jax.ShapeDtypeStruct((CAP, D), jnp.float32),
    )(bexp, gathered, W1, W3, W2, wsort2)


def _scatter_body(rs_hbm, dst_hbm, out_hbm, idx_v, rows_v, sem):
    wid = lax.axis_index("s") * NC + lax.axis_index("c")
    base = wid * RPW
    for c in range(RPW // RCH):
        off = base + c * RCH
        pltpu.sync_copy(dst_hbm.at[pl.ds(off, RCH)], idx_v)
        pltpu.sync_copy(rs_hbm.at[pl.ds(off, RCH)], rows_v)
        pltpu.async_copy(rows_v, out_hbm.at[idx_v], sem).wait()


_scatter = functools.partial(
    pl.kernel,
    _scatter_body,
    out_type=jax.ShapeDtypeStruct((AK + 8, D), jnp.float32),
    mesh=plsc.VectorSubcoreMesh(core_axis_name="c", subcore_axis_name="s", num_cores=NC, num_subcores=NS),
    compiler_params=pltpu.CompilerParams(needs_layout_passes=False),
    scratch_types=[
        pltpu.VMEM((RCH,), jnp.int32),
        pltpu.VMEM((RCH, D), jnp.float32),
        pltpu.SemaphoreType.DMA,
    ],
)()


def _combine_body(y_ref, a_ref, b_ref, out_ref):
    out_ref[...] = y_ref[...] + a_ref[...] + b_ref[...]


def _combine(shared_y, slotbuf):
    return pl.pallas_call(
        _combine_body,
        grid=(NT,),
        in_specs=[
            pl.BlockSpec((TBLK, D), lambda t: (t, 0)),
            pl.BlockSpec((TBLK, D), lambda t: (t, 0)),
            pl.BlockSpec((TBLK, D), lambda t: (t + NT, 0)),
        ],
        out_specs=pl.BlockSpec((TBLK, D), lambda t: (t, 0)),
        out_shape=jax.ShapeDtypeStruct((T, D), jnp.float32),
    )(shared_y, slotbuf, slotbuf)


@jax.jit
def _moe(flat, gate_w, W1, W2, W3, sw1, sw2, sw3):
    shared_y, eid2, w2sc, hist = _route(flat, gate_w, sw1, sw2, sw3)
    eidflat = eid2.reshape(AK)
    wflat = w2sc.reshape(AK)
    hist16 = hist.reshape(2 * E)
    perm, dst, wsort, bexp = _sort(eidflat, wflat, hist16)
    gathered = _gather(flat, perm)
    routed_sorted = _ffn(bexp, gathered, W1, W3, W2, wsort.reshape(CAP, 1))
    slotbuf = _scatter(routed_sorted, dst)
    return _combine(shared_y, slotbuf)


def kernel(x, input_ids, gate_w, W1, W2, W3, sw1, sw2, sw3):
    del input_ids
    flat = x.reshape(-1, D)
    out = _moe(flat, gate_w, W1, W2, W3, sw1, sw2, sw3)
    return out.reshape(x.shape)
